# baseline (speedup 1.0000x reference)
import jax
import jax.numpy as jnp
from jax.experimental import pallas as pl
from jax.experimental.pallas import tpu as pltpu

_N_CHUNKS = 4


def _fused_gcn(emb_ref, adj_hbm_ref, w1_ref, b1_ref, w2_ref, b2_ref, out_ref,
               adj_f32_ref, adj_bf_ref, sems):
    n = out_ref.shape[0]
    chunk = n // _N_CHUNKS

    copies = [
        pltpu.make_async_copy(
            adj_hbm_ref.at[pl.ds(c * chunk, chunk), :],
            adj_f32_ref.at[pl.ds(c * chunk, chunk), :],
            sems.at[c])
        for c in range(_N_CHUNKS)
    ]
    for cp in copies:
        cp.start()

    xw1 = jnp.dot(emb_ref[...], w1_ref[...],
                  preferred_element_type=jnp.float32)

    deg_row = jnp.zeros((1, n), dtype=jnp.float32)
    for c, cp in enumerate(copies):
        cp.wait()
        blk = adj_f32_ref[pl.ds(c * chunk, chunk), :]
        deg_row = deg_row + jnp.sum(blk, axis=0, keepdims=True)
        adj_bf_ref[pl.ds(c * chunk, chunk), :] = blk.astype(jnp.bfloat16)

    dinv = jnp.transpose(jax.lax.rsqrt(deg_row + 1.0))
    adj = adj_bf_ref[...]

    s1 = xw1 * dinv
    t1 = jax.lax.dot_general(
        adj, s1.astype(jnp.bfloat16), (((0,), (0,)), ((), ())),
        preferred_element_type=jnp.float32) + s1
    h1 = jnp.maximum(t1 * dinv + b1_ref[...], 0.0)

    s2 = jnp.dot(h1, w2_ref[...], preferred_element_type=jnp.float32) * dinv
    t2 = jax.lax.dot_general(
        adj, s2.astype(jnp.bfloat16), (((0,), (0,)), ((), ())),
        preferred_element_type=jnp.float32) + s2
    out_ref[...] = t2 * dinv + b2_ref[...]


def kernel(emb, adj, W1, b1, W2, b2):
    n = emb.shape[0]
    hidden = W1.shape[1]
    out_dim = W2.shape[1]
    vmem = pl.BlockSpec(memory_space=pltpu.MemorySpace.VMEM)
    return pl.pallas_call(
        _fused_gcn,
        in_specs=[
            vmem,
            pl.BlockSpec(memory_space=pltpu.MemorySpace.HBM),
            vmem, vmem, vmem, vmem,
        ],
        out_specs=vmem,
        out_shape=jax.ShapeDtypeStruct((n, out_dim), jnp.float32),
        scratch_shapes=[
            pltpu.VMEM((n, n), jnp.float32),
            pltpu.VMEM((n, n), jnp.bfloat16),
            pltpu.SemaphoreType.DMA((_N_CHUNKS,)),
        ],
    )(emb, adj, W1, b1.reshape(1, -1), W2, b2.reshape(1, -1))


# speedup vs baseline: 1.0060x; 1.0060x over previous
import jax
import jax.numpy as jnp
from jax.experimental import pallas as pl
from jax.experimental.pallas import tpu as pltpu

_N_CHUNKS = 4


def _fused_gcn(emb_ref, adj_hbm_ref, w1_ref, b1_ref, w2_ref, b2_ref, out_ref,
               adj_f32_ref, sems):
    n = out_ref.shape[0]

    cp = pltpu.make_async_copy(adj_hbm_ref, adj_f32_ref, sems.at[0])
    cp.start()

    xw1 = jnp.dot(emb_ref[...], w1_ref[...],
                  preferred_element_type=jnp.float32)

    cp.wait()
    adj = adj_f32_ref[...].astype(jnp.bfloat16)

    deg_row = jnp.sum(adj_f32_ref[...], axis=0, keepdims=True) + 1.0
    dinv = jnp.transpose(jax.lax.rsqrt(deg_row))

    s1 = xw1 * dinv
    t1 = jax.lax.dot_general(
        adj, s1.astype(jnp.bfloat16), (((0,), (0,)), ((), ())),
        preferred_element_type=jnp.float32) + s1
    h1 = jnp.maximum(t1 * dinv + b1_ref[...], 0.0)

    s2 = jnp.dot(h1, w2_ref[...], preferred_element_type=jnp.float32) * dinv
    t2 = jax.lax.dot_general(
        adj, s2.astype(jnp.bfloat16), (((0,), (0,)), ((), ())),
        preferred_element_type=jnp.float32) + s2
    out_ref[...] = t2 * dinv + b2_ref[...]


def kernel(emb, adj, W1, b1, W2, b2):
    n = emb.shape[0]
    hidden = W1.shape[1]
    out_dim = W2.shape[1]
    vmem = pl.BlockSpec(memory_space=pltpu.MemorySpace.VMEM)
    return pl.pallas_call(
        _fused_gcn,
        in_specs=[
            vmem,
            pl.BlockSpec(memory_space=pltpu.MemorySpace.HBM),
            vmem, vmem, vmem, vmem,
        ],
        out_specs=vmem,
        out_shape=jax.ShapeDtypeStruct((n, out_dim), jnp.float32),
        scratch_shapes=[
            pltpu.VMEM((n, n), jnp.float32),
            pltpu.SemaphoreType.DMA((1,)),
        ],
    )(emb, adj, W1, b1.reshape(1, -1), W2, b2.reshape(1, -1))


# speedup vs baseline: 1.1340x; 1.1272x over previous
import jax
import jax.numpy as jnp
from jax.experimental import pallas as pl


def _fused_gcn(emb_ref, adj_ref, w1_ref, b1_ref, w2_ref, b2_ref, out_ref):
    adj = adj_ref[...].astype(jnp.bfloat16)
    n = adj.shape[0]

    deg_row = jnp.sum(adj_ref[...], axis=0, keepdims=True) + 1.0
    dinv = jnp.transpose(jax.lax.rsqrt(deg_row))

    s1 = jnp.dot(emb_ref[...], w1_ref[...],
                 preferred_element_type=jnp.float32) * dinv
    t1 = jax.lax.dot_general(
        adj, s1.astype(jnp.bfloat16), (((0,), (0,)), ((), ())),
        preferred_element_type=jnp.float32) + s1
    h1 = jnp.maximum(t1 * dinv + b1_ref[...], 0.0)

    s2 = jnp.dot(h1, w2_ref[...], preferred_element_type=jnp.float32) * dinv
    t2 = jax.lax.dot_general(
        adj, s2.astype(jnp.bfloat16), (((0,), (0,)), ((), ())),
        preferred_element_type=jnp.float32) + s2
    out_ref[...] = t2 * dinv + b2_ref[...]


def kernel(emb, adj, W1, b1, W2, b2):
    n = emb.shape[0]
    out_dim = W2.shape[1]
    return pl.pallas_call(
        _fused_gcn,
        out_shape=jax.ShapeDtypeStruct((n, out_dim), jnp.float32),
    )(emb, adj, W1, b1.reshape(1, -1), W2, b2.reshape(1, -1))
